# baseline (speedup 1.0000x reference)
import jax
import jax.numpy as jnp
from jax.experimental import pallas as pl

MASK_TOKEN = 103
VOCAB = 30522
MULT = (2 ** 16 % VOCAB) ** 2 % VOCAB
W16 = 2 ** 16 % VOCAB
RECIP = 1.0 / VOCAB

K_RAND = (1832780943, 270669613)
K_HI = (3187376881, 129218101)
K_LO = (2350016172, 1168365246)

_ROT_A = (13, 15, 26, 6)
_ROT_B = (17, 29, 16, 24)


def _rotl(x, d):
    return jax.lax.shift_left(x, jnp.uint32(d)) | jax.lax.shift_right_logical(
        x, jnp.uint32(32 - d))


def _threefry_bits(k1, k2, idx):
    ks0 = jnp.uint32(k1)
    ks1 = jnp.uint32(k2)
    ks2 = jnp.uint32(k1 ^ k2 ^ 0x1BD11BDA)
    ks = (ks0, ks1, ks2)
    x0 = jnp.full_like(idx, ks0)
    x1 = idx + ks1
    rots = (_ROT_A, _ROT_B, _ROT_A, _ROT_B, _ROT_A)
    for i in range(5):
        for r in rots[i]:
            x0 = x0 + x1
            x1 = _rotl(x1, r)
            x1 = x0 ^ x1
        x0 = x0 + ks[(i + 1) % 3]
        x1 = x1 + ks[(i + 2) % 3] + jnp.uint32(i + 1)
    return x0 ^ x1


def _mod_span(t):
    q = jnp.floor(t.astype(jnp.float32) * jnp.float32(RECIP)).astype(jnp.int32)
    r = t - q * jnp.int32(VOCAB)
    r = jnp.where(r < 0, r + jnp.int32(VOCAB), r)
    r = jnp.where(r >= jnp.int32(VOCAB), r - jnp.int32(VOCAB), r)
    return r


def _mod_span_u32(bits):
    a = jax.lax.shift_right_logical(bits, jnp.uint32(16)).astype(jnp.int32)
    b = (bits & jnp.uint32(0xFFFF)).astype(jnp.int32)
    return _mod_span(a * jnp.int32(W16) + b)


def _uniform01(bits):
    fb = jax.lax.shift_right_logical(bits, jnp.uint32(9)) | jnp.uint32(0x3F800000)
    return jax.lax.bitcast_convert_type(fb, jnp.float32) - jnp.float32(1.0)


def _mask_kernel(x_ref, out_ref, lab_ref):
    x = x_ref[...]
    rows, cols = x.shape
    row = jax.lax.broadcasted_iota(jnp.uint32, (rows, cols), 0)
    col = jax.lax.broadcasted_iota(jnp.uint32, (rows, cols), 1)
    idx = row * jnp.uint32(cols) + col

    rand = _uniform01(_threefry_bits(*K_RAND, idx))
    masked = rand < jnp.float32(0.15)
    mask_mask = masked & (rand < jnp.float32(0.15 * 0.8))
    random_mask = (masked & (rand >= jnp.float32(0.15 * 0.8))
                   & (rand < jnp.float32(0.15 * 0.9)))

    hi = _threefry_bits(*K_HI, idx)
    lo = _threefry_bits(*K_LO, idx)
    toks = _mod_span(_mod_span_u32(hi) * jnp.int32(MULT) + _mod_span_u32(lo))

    lab_ref[...] = jnp.where(masked, x, jnp.int32(-100))
    out = jnp.where(mask_mask, jnp.int32(MASK_TOKEN), x)
    out_ref[...] = jnp.where(random_mask, toks, out)


def kernel(x):
    shape = x.shape
    n = shape[0] * shape[1]
    x2 = x.reshape(n // 4096, 4096)
    out_shape = jax.ShapeDtypeStruct(x2.shape, x.dtype)
    out, lab = pl.pallas_call(
        _mask_kernel,
        out_shape=(out_shape, out_shape),
    )(x2)
    return out.reshape(shape), lab.reshape(shape)


# speedup vs baseline: 1.9612x; 1.9612x over previous
import jax
import jax.numpy as jnp
from jax.experimental import pallas as pl

MASK_TOKEN = 103
VOCAB = 30522
MULT = (2 ** 16 % VOCAB) ** 2 % VOCAB
W16 = 2 ** 16 % VOCAB
RECIP = 1.0 / VOCAB

K_RAND = (1832780943, 270669613)
K_HI = (3187376881, 129218101)
K_LO = (2350016172, 1168365246)

_ROT_A = (13, 15, 26, 6)
_ROT_B = (17, 29, 16, 24)


def _rotl(x, d):
    return jax.lax.shift_left(x, jnp.uint32(d)) | jax.lax.shift_right_logical(
        x, jnp.uint32(32 - d))


def _threefry_bits(k1, k2, idx):
    ks0 = jnp.uint32(k1)
    ks1 = jnp.uint32(k2)
    ks2 = jnp.uint32(k1 ^ k2 ^ 0x1BD11BDA)
    ks = (ks0, ks1, ks2)
    x0 = jnp.full_like(idx, ks0)
    x1 = idx + ks1
    rots = (_ROT_A, _ROT_B, _ROT_A, _ROT_B, _ROT_A)
    for i in range(5):
        for r in rots[i]:
            x0 = x0 + x1
            x1 = _rotl(x1, r)
            x1 = x0 ^ x1
        x0 = x0 + ks[(i + 1) % 3]
        x1 = x1 + ks[(i + 2) % 3] + jnp.uint32(i + 1)
    return x0 ^ x1


def _mod_span(t):
    q = jnp.floor(t.astype(jnp.float32) * jnp.float32(RECIP)).astype(jnp.int32)
    r = t - q * jnp.int32(VOCAB)
    r = jnp.where(r < 0, r + jnp.int32(VOCAB), r)
    r = jnp.where(r >= jnp.int32(VOCAB), r - jnp.int32(VOCAB), r)
    return r


def _mod_span_u32(bits):
    a = jax.lax.shift_right_logical(bits, jnp.uint32(16)).astype(jnp.int32)
    b = (bits & jnp.uint32(0xFFFF)).astype(jnp.int32)
    return _mod_span(a * jnp.int32(W16) + b)


def _uniform01(bits):
    fb = jax.lax.shift_right_logical(bits, jnp.uint32(9)) | jnp.uint32(0x3F800000)
    return jax.lax.bitcast_convert_type(fb, jnp.float32) - jnp.float32(1.0)


def _mask_kernel(x_ref, out_ref, lab_ref):
    rows, cols = x_ref.shape
    half = cols // 2
    row = jax.lax.broadcasted_iota(jnp.uint32, (2 * rows, half), 0)
    col = jax.lax.broadcasted_iota(jnp.uint32, (2 * rows, half), 1)
    idx = ((row & jnp.uint32(3)) * jnp.uint32(cols)
           + jax.lax.shift_right_logical(row, jnp.uint32(2)) * jnp.uint32(half)
           + col)

    rand = _uniform01(_threefry_bits(*K_RAND, idx))
    masked = rand < jnp.float32(0.15)
    mask_mask = rand < jnp.float32(0.15 * 0.8)
    random_mask = masked & (rand >= jnp.float32(0.15 * 0.8)) & (
        rand < jnp.float32(0.15 * 0.9))

    hi = _threefry_bits(*K_HI, idx)
    lo = _threefry_bits(*K_LO, idx)
    toks = _mod_span(_mod_span_u32(hi) * jnp.int32(MULT) + _mod_span_u32(lo))

    code = jnp.where(masked,
                     jnp.where(mask_mask, jnp.int32(MASK_TOKEN),
                               jnp.where(random_mask, toks, jnp.int32(-1))),
                     jnp.int32(-2))

    for h in range(2):
        c = code[h * rows:(h + 1) * rows, :]
        xs = x_ref[:, h * half:(h + 1) * half]
        lab_ref[:, h * half:(h + 1) * half] = jnp.where(
            c == jnp.int32(-2), jnp.int32(-100), xs)
        out_ref[:, h * half:(h + 1) * half] = jnp.where(
            c >= jnp.int32(0), c, xs)


def kernel(x):
    out_shape = jax.ShapeDtypeStruct(x.shape, x.dtype)
    return pl.pallas_call(
        _mask_kernel,
        out_shape=(out_shape, out_shape),
    )(x)
